# baseline (speedup 1.0000x reference)
import jax
import jax.numpy as jnp
from jax.experimental import pallas as pl
from jax.experimental.pallas import tpu as pltpu

_ALPHA = 0.1
_BETA = 0.5

_BM = 80
_NCHUNK = 5


def _make_body(nchunk, bm):
    def body(*refs):
        g_refs = refs[:nchunk]
        f_ref, f0_ref, w_ref, b_ref, o_ref = refs[nchunk:]
        f = f_ref[...]
        w = w_ref[...]
        bias = b_ref[...]
        for c, g_ref in enumerate(g_refs):
            hi = jnp.dot(g_ref[...], f, preferred_element_type=jnp.float32)
            support = (1.0 - _ALPHA) * hi + _ALPHA * f0_ref[c * bm:(c + 1) * bm, :]
            out = _BETA * jnp.dot(support, w, preferred_element_type=jnp.float32)
            o_ref[c * bm:(c + 1) * bm, :] = out + (1.0 - _BETA) * support + bias

    return body


def kernel(graph, features, features0, w, b):
    n, k = graph.shape
    f = features.shape[1]
    fo = w.shape[1]
    b2 = b.reshape(1, fo)

    bm, nchunk = (_BM, _NCHUNK) if n % (_BM * _NCHUNK) == 0 else (n, 1)
    rows_per_step = bm * nchunk
    grid = (n // rows_per_step,)

    def g_spec(c):
        return pl.BlockSpec((bm, k), lambda i, c=c: (i * nchunk + c, 0))

    return pl.pallas_call(
        _make_body(nchunk, bm),
        grid=grid,
        in_specs=[g_spec(c) for c in range(nchunk)] + [
            pl.BlockSpec((k, f), lambda i: (0, 0)),
            pl.BlockSpec((rows_per_step, f), lambda i: (i, 0)),
            pl.BlockSpec((f, fo), lambda i: (0, 0)),
            pl.BlockSpec((1, fo), lambda i: (0, 0)),
        ],
        out_specs=pl.BlockSpec((rows_per_step, fo), lambda i: (i, 0)),
        out_shape=jax.ShapeDtypeStruct((n, fo), jnp.float32),
        compiler_params=pltpu.CompilerParams(
            dimension_semantics=("parallel",),
        ),
    )(*([graph] * nchunk), features, features0, w, b2)


# speedup vs baseline: 1.4001x; 1.4001x over previous
import jax
import jax.numpy as jnp
from jax.experimental import pallas as pl
from jax.experimental.pallas import tpu as pltpu

_ALPHA = 0.1
_BETA = 0.5

_BM = 200
_RING = 5


def _make_manual_body(nblocks, nrounds):
    def body(g_hbm, f_hbm, f0_hbm, w_ref, b_ref, o_hbm, *scratch):
        g_bufs = scratch[0:_RING]
        f0_bufs = scratch[_RING:2 * _RING]
        o_bufs = scratch[2 * _RING:3 * _RING]
        f_vmem = scratch[3 * _RING]
        g_sem, f0_sem, o_sem, f_sem = scratch[3 * _RING + 1:]

        def g_copy(j, s):
            return pltpu.make_async_copy(
                g_hbm.at[pl.ds(j * _BM, _BM), :], g_bufs[s], g_sem.at[s])

        def f0_copy(j, s):
            return pltpu.make_async_copy(
                f0_hbm.at[pl.ds(j * _BM, _BM), :], f0_bufs[s], f0_sem.at[s])

        def o_copy(j, s):
            return pltpu.make_async_copy(
                o_bufs[s], o_hbm.at[pl.ds(j * _BM, _BM), :], o_sem.at[s])

        for s in range(_RING):
            g_copy(s, s).start()
            f0_copy(s, s).start()
        pltpu.make_async_copy(f_hbm, f_vmem, f_sem).start()

        def round_step(r, carry):
            for s in range(_RING):
                j = r * _RING + s
                g_copy(j, s).wait()
                if s == 0:
                    @pl.when(r == 0)
                    def _wait_f():
                        pltpu.make_async_copy(f_hbm, f_vmem, f_sem).wait()
                hi = jnp.dot(g_bufs[s][...], f_vmem[...],
                             preferred_element_type=jnp.float32)
                f0_copy(j, s).wait()
                support = (1.0 - _ALPHA) * hi + _ALPHA * f0_bufs[s][...]
                out = _BETA * jnp.dot(support, w_ref[...],
                                      preferred_element_type=jnp.float32)
                out = out + (1.0 - _BETA) * support + b_ref[...]

                @pl.when(r > 0)
                def _recycle_out():
                    o_copy(j - _RING, s).wait()

                o_bufs[s][...] = out
                o_copy(j, s).start()

                @pl.when(r < nrounds - 1)
                def _refill():
                    g_copy(j + _RING, s).start()
                    f0_copy(j + _RING, s).start()
            return carry

        jax.lax.fori_loop(0, nrounds, round_step, 0)
        for s in range(_RING):
            o_copy(nblocks - _RING + s, s).wait()

    return body


def _manual_kernel(graph, features, features0, w, b2):
    n, k = graph.shape
    f = features.shape[1]
    fo = w.shape[1]
    nblocks = n // _BM
    nrounds = nblocks // _RING

    return pl.pallas_call(
        _make_manual_body(nblocks, nrounds),
        in_specs=[
            pl.BlockSpec(memory_space=pltpu.MemorySpace.HBM),
            pl.BlockSpec(memory_space=pltpu.MemorySpace.HBM),
            pl.BlockSpec(memory_space=pltpu.MemorySpace.HBM),
            pl.BlockSpec(memory_space=pltpu.MemorySpace.VMEM),
            pl.BlockSpec(memory_space=pltpu.MemorySpace.VMEM),
        ],
        out_specs=pl.BlockSpec(memory_space=pltpu.MemorySpace.HBM),
        out_shape=jax.ShapeDtypeStruct((n, fo), jnp.float32),
        scratch_shapes=(
            [pltpu.VMEM((_BM, k), jnp.float32) for _ in range(_RING)]
            + [pltpu.VMEM((_BM, f), jnp.float32) for _ in range(_RING)]
            + [pltpu.VMEM((_BM, fo), jnp.float32) for _ in range(_RING)]
            + [
                pltpu.VMEM((k, f), jnp.float32),
                pltpu.SemaphoreType.DMA((_RING,)),
                pltpu.SemaphoreType.DMA((_RING,)),
                pltpu.SemaphoreType.DMA((_RING,)),
                pltpu.SemaphoreType.DMA,
            ]
        ),
    )(graph, features, features0, w, b2)


def _auto_body(g_ref, f_ref, f0_ref, w_ref, b_ref, o_ref):
    hi = jnp.dot(g_ref[...], f_ref[...], preferred_element_type=jnp.float32)
    support = (1.0 - _ALPHA) * hi + _ALPHA * f0_ref[...]
    out = _BETA * jnp.dot(support, w_ref[...], preferred_element_type=jnp.float32)
    o_ref[...] = out + (1.0 - _BETA) * support + b_ref[...]


def _auto_kernel(graph, features, features0, w, b2):
    n, k = graph.shape
    f = features.shape[1]
    fo = w.shape[1]
    bm = 400 if n % 400 == 0 else n
    grid = (n // bm,)
    return pl.pallas_call(
        _auto_body,
        grid=grid,
        in_specs=[
            pl.BlockSpec((bm, k), lambda i: (i, 0)),
            pl.BlockSpec((k, f), lambda i: (0, 0)),
            pl.BlockSpec((bm, f), lambda i: (i, 0)),
            pl.BlockSpec((f, fo), lambda i: (0, 0)),
            pl.BlockSpec((1, fo), lambda i: (0, 0)),
        ],
        out_specs=pl.BlockSpec((bm, fo), lambda i: (i, 0)),
        out_shape=jax.ShapeDtypeStruct((n, fo), jnp.float32),
        compiler_params=pltpu.CompilerParams(
            dimension_semantics=("parallel",),
        ),
    )(graph, features, features0, w, b2)


def kernel(graph, features, features0, w, b):
    n = graph.shape[0]
    fo = w.shape[1]
    b2 = b.reshape(1, fo)
    if n % (_BM * _RING) == 0:
        return _manual_kernel(graph, features, features0, w, b2)
    return _auto_kernel(graph, features, features0, w, b2)


# speedup vs baseline: 1.4391x; 1.0279x over previous
import jax
import jax.numpy as jnp
from jax.experimental import pallas as pl
from jax.experimental.pallas import tpu as pltpu

_ALPHA = 0.1
_BETA = 0.5

_BM = 200
_RING = 5


def _make_manual_body(nblocks, nrounds):
    def body(g_hbm, f_hbm, f0_hbm, w_ref, b_ref, o_hbm, *scratch):
        g_bufs = scratch[0:_RING]
        f0_bufs = scratch[_RING:2 * _RING]
        o_bufs = scratch[2 * _RING:3 * _RING]
        f_vmem = scratch[3 * _RING]
        g_sem, f0_sem, o_sem, f_sem = scratch[3 * _RING + 1:]

        def g_copy(j, s):
            return pltpu.make_async_copy(
                g_hbm.at[pl.ds(j * _BM, _BM), :], g_bufs[s], g_sem.at[s])

        def f0_copy(j, s):
            return pltpu.make_async_copy(
                f0_hbm.at[pl.ds(j * _BM, _BM), :], f0_bufs[s], f0_sem.at[s])

        def o_copy(j, s):
            return pltpu.make_async_copy(
                o_bufs[s], o_hbm.at[pl.ds(j * _BM, _BM), :], o_sem.at[s])

        pltpu.make_async_copy(f_hbm, f_vmem, f_sem).start()
        for s in range(_RING):
            g_copy(s, s).start()
            f0_copy(s, s).start()

        def round_step(r, carry):
            for s in range(_RING):
                j = r * _RING + s
                g_copy(j, s).wait()
                if s == 0:
                    @pl.when(r == 0)
                    def _wait_f():
                        pltpu.make_async_copy(f_hbm, f_vmem, f_sem).wait()
                hi = jnp.dot(g_bufs[s][...], f_vmem[...],
                             preferred_element_type=jnp.float32)
                f0_copy(j, s).wait()
                support = (1.0 - _ALPHA) * hi + _ALPHA * f0_bufs[s][...]
                out = _BETA * jnp.dot(support, w_ref[...],
                                      preferred_element_type=jnp.float32)
                out = out + (1.0 - _BETA) * support + b_ref[...]

                @pl.when(r > 0)
                def _recycle_out():
                    o_copy(j - _RING, s).wait()

                o_bufs[s][...] = out
                o_copy(j, s).start()

                @pl.when(r < nrounds - 1)
                def _refill():
                    g_copy(j + _RING, s).start()
                    f0_copy(j + _RING, s).start()
            return carry

        jax.lax.fori_loop(0, nrounds, round_step, 0)
        for s in range(_RING):
            o_copy(nblocks - _RING + s, s).wait()

    return body


def _manual_kernel(graph, features, features0, w, b2):
    n, k = graph.shape
    f = features.shape[1]
    fo = w.shape[1]
    nblocks = n // _BM
    nrounds = nblocks // _RING

    return pl.pallas_call(
        _make_manual_body(nblocks, nrounds),
        in_specs=[
            pl.BlockSpec(memory_space=pltpu.MemorySpace.HBM),
            pl.BlockSpec(memory_space=pltpu.MemorySpace.HBM),
            pl.BlockSpec(memory_space=pltpu.MemorySpace.HBM),
            pl.BlockSpec(memory_space=pltpu.MemorySpace.VMEM),
            pl.BlockSpec(memory_space=pltpu.MemorySpace.VMEM),
        ],
        out_specs=pl.BlockSpec(memory_space=pltpu.MemorySpace.HBM),
        out_shape=jax.ShapeDtypeStruct((n, fo), jnp.float32),
        scratch_shapes=(
            [pltpu.VMEM((_BM, k), jnp.float32) for _ in range(_RING)]
            + [pltpu.VMEM((_BM, f), jnp.float32) for _ in range(_RING)]
            + [pltpu.VMEM((_BM, fo), jnp.float32) for _ in range(_RING)]
            + [
                pltpu.VMEM((k, f), jnp.float32),
                pltpu.SemaphoreType.DMA((_RING,)),
                pltpu.SemaphoreType.DMA((_RING,)),
                pltpu.SemaphoreType.DMA((_RING,)),
                pltpu.SemaphoreType.DMA,
            ]
        ),
    )(graph, features, features0, w, b2)


def _auto_body(g_ref, f_ref, f0_ref, w_ref, b_ref, o_ref):
    hi = jnp.dot(g_ref[...], f_ref[...], preferred_element_type=jnp.float32)
    support = (1.0 - _ALPHA) * hi + _ALPHA * f0_ref[...]
    out = _BETA * jnp.dot(support, w_ref[...], preferred_element_type=jnp.float32)
    o_ref[...] = out + (1.0 - _BETA) * support + b_ref[...]


def _auto_kernel(graph, features, features0, w, b2):
    n, k = graph.shape
    f = features.shape[1]
    fo = w.shape[1]
    bm = 400 if n % 400 == 0 else n
    grid = (n // bm,)
    return pl.pallas_call(
        _auto_body,
        grid=grid,
        in_specs=[
            pl.BlockSpec((bm, k), lambda i: (i, 0)),
            pl.BlockSpec((k, f), lambda i: (0, 0)),
            pl.BlockSpec((bm, f), lambda i: (i, 0)),
            pl.BlockSpec((f, fo), lambda i: (0, 0)),
            pl.BlockSpec((1, fo), lambda i: (0, 0)),
        ],
        out_specs=pl.BlockSpec((bm, fo), lambda i: (i, 0)),
        out_shape=jax.ShapeDtypeStruct((n, fo), jnp.float32),
        compiler_params=pltpu.CompilerParams(
            dimension_semantics=("parallel",),
        ),
    )(graph, features, features0, w, b2)


def kernel(graph, features, features0, w, b):
    n = graph.shape[0]
    fo = w.shape[1]
    b2 = b.reshape(1, fo)
    if n % (_BM * _RING) == 0:
        return _manual_kernel(graph, features, features0, w, b2)
    return _auto_kernel(graph, features, features0, w, b2)
